# baseline (speedup 1.0000x reference)
import jax
import jax.numpy as jnp
from jax import lax
from jax.experimental import pallas as pl
from jax.experimental.pallas import tpu as pltpu
from jax.experimental.pallas import tpu_sc as plsc

_NC = 2
_NS = 16
_NW = _NC * _NS


def _gather_linear(token_indices, lookup):
    B, S = token_indices.shape
    V, D = lookup.shape
    N = B * S
    b_per_w = N // _NW
    NBUF = 4
    C = 800
    n_chunks = b_per_w // C
    assert n_chunks % NBUF == 0 and n_chunks >= NBUF

    idx = token_indices.reshape(N).astype(jnp.int32)
    mesh = plsc.VectorSubcoreMesh(core_axis_name="core", subcore_axis_name="subcore")

    @pl.kernel(
        out_type=jax.ShapeDtypeStruct((N, D), lookup.dtype),
        mesh=mesh,
        compiler_params=pltpu.CompilerParams(use_tc_tiling_on_sc=False),
        scratch_types=(
            [pltpu.VMEM((b_per_w,), jnp.int32),
             pltpu.VMEM((NBUF, C, D), lookup.dtype)]
            + [pltpu.SemaphoreType.DMA] * (1 + 2 * NBUF)
        ),
    )
    def gather_kernel(table_hbm, idx_hbm, out_hbm, idx_v, rows_v, isem, *sems):
        gsem = sems[:NBUF]
        osem = sems[NBUF:]
        wid = lax.axis_index("subcore") * _NC + lax.axis_index("core")
        base = wid * b_per_w
        pltpu.async_copy(idx_hbm.at[pl.ds(base, b_per_w)], idx_v, isem).wait()

        def g_copy(g, b):
            return pltpu.make_async_copy(
                table_hbm.at[idx_v.at[pl.ds(g * C, C)]], rows_v.at[b], gsem[b])

        def o_copy(g, b):
            return pltpu.make_async_copy(
                rows_v.at[b], out_hbm.at[pl.ds(base + g * C, C)], osem[b])

        for b in range(NBUF):
            g_copy(b, b).start()

        @pl.loop(0, n_chunks, step=NBUF)
        def _(gi):
            for b in range(NBUF):
                g = gi + b
                g_copy(g, b).wait()
                o_copy(g, b).start()
                nxt = g + NBUF

                @pl.when(nxt < n_chunks)
                def _():
                    o_copy(g, b).wait()
                    g_copy(nxt, b).start()

        for b in range(NBUF):
            o_copy(n_chunks - NBUF + b, b).wait()

    return gather_kernel(lookup, idx)


def _expand_tiled(lin, B, S, D):
    LPB = S * D // 128
    TB = B // _NW
    NCH = TB // 4
    mesh = plsc.VectorSubcoreMesh(core_axis_name="core", subcore_axis_name="subcore")

    @pl.kernel(
        out_type=jax.ShapeDtypeStruct((B, S, D), jnp.float32),
        mesh=mesh,
        scratch_types=(
            [pltpu.VMEM((2, 4 * LPB, 128), jnp.float32),
             pltpu.VMEM((2, S, D), jnp.float32)]
            + [pltpu.SemaphoreType.DMA] * 4
        ),
    )
    def expand_kernel(lin_hbm, out_hbm, lbuf, pbuf, *sems):
        isem = sems[:2]
        osem = sems[2:]
        wid = lax.axis_index("subcore") * _NC + lax.axis_index("core")
        b0 = wid * TB

        def i_copy(ch, ib):
            return pltpu.make_async_copy(
                lin_hbm.at[pl.ds((b0 + 4 * ch) * LPB, 4 * LPB)], lbuf.at[ib],
                isem[ib])

        def o_copy(bb, pb):
            return pltpu.make_async_copy(pbuf.at[pb], out_hbm.at[bb], osem[pb])

        i_copy(0, 0).start()

        @pl.loop(0, NCH, step=2)
        def _(ci):
            for ib in range(2):
                ch = ci + ib
                i_copy(ch, ib).wait()

                @pl.when(ch + 1 < NCH)
                def _():
                    i_copy(ch + 1, 1 - ib).start()

                for u in range(4):
                    pb = u % 2
                    bb = b0 + 4 * ch + u
                    if u < 2:
                        @pl.when(ch > 0)
                        def _():
                            o_copy(bb - 2, pb).wait()
                    else:
                        o_copy(bb - 2, pb).wait()

                    @pl.loop(0, LPB)
                    def _(lr):
                        for v in range(4):
                            for h in range(D // 16):
                                pbuf[pb, 4 * lr + v, pl.ds(16 * h, 16)] = (
                                    lbuf[ib, LPB * u + lr,
                                         pl.ds(32 * v + 16 * h, 16)])

                    o_copy(bb, pb).start()

        o_copy(b0 + TB - 2, 0).wait()
        o_copy(b0 + TB - 1, 1).wait()

    return expand_kernel(lin)


def kernel(token_indices, lookup):
    if token_indices.ndim == 1:
        token_indices = token_indices[None, :]
    B, S = token_indices.shape
    V, D = lookup.shape
    N = B * S
    flat = _gather_linear(token_indices, lookup)
    lin = flat.reshape(N * D // 128, 128)
    return _expand_tiled(lin, B, S, D)
